# baseline (speedup 1.0000x reference)
import functools

import jax
import jax.numpy as jnp
from jax import lax
from jax.experimental import pallas as pl
from jax.experimental.pallas import tpu as pltpu
from jax.experimental.pallas import tpu_sc as plsc

B, S, E = 4096, 50, 128
H, O = 64, 10
NC, NS = 2, 16
NW = NC * NS
BPW = B // NW
PAIR = 1
CSZ = PAIR * S
CH = BPW // PAIR
EV = E // 16
NBUF = 8


def _rowsum(bufslot, base):
    UNR = 5

    def jb(j, carry):
        r = base + j * UNR
        vs = list(carry)
        for u in range(UNR):
            for e in range(EV):
                vs[e] = vs[e] + bufslot[r + u, pl.ds(e * 16, 16)]
        return tuple(vs)

    z = tuple(jnp.zeros((16,), jnp.float32) for _ in range(EV))
    return lax.fori_loop(0, S // UNR, jb, z)


def _sc_gather_sum(xg, table, dummy):
    bh = xg.shape[0]
    bpw = bh // NW
    ch = bpw // PAIR
    mesh = plsc.VectorSubcoreMesh(core_axis_name="c", subcore_axis_name="s")

    @functools.partial(
        pl.kernel,
        out_type=jax.ShapeDtypeStruct((NW, bpw, E), jnp.float32),
        mesh=mesh,
        scratch_types=[
            pltpu.VMEM((ch, CSZ), jnp.int32),
            pltpu.VMEM((NBUF, CSZ, E), jnp.float32),
            pltpu.VMEM((bpw, E), jnp.float32),
            [pltpu.SemaphoreType.DMA] * NBUF,
        ],
    )
    def run(x_hbm, table_hbm, dummy_hbm, out_hbm, xv, buf, acc, sems):
        wid = lax.axis_index("s") * NC + lax.axis_index("c")
        pltpu.sync_copy(x_hbm.at[pl.ds(wid * bpw, bpw)], xv)
        for slot in range(NBUF):
            pltpu.async_copy(table_hbm.at[xv.at[slot]], buf.at[slot], sems[slot])

        def step(slot, ci):
            pltpu.make_async_copy(
                dummy_hbm, buf.at[slot], sems[slot]
            ).wait()
            for p in range(PAIR):
                vs = _rowsum(buf.at[slot], p * S)
                row = ci * PAIR + p
                for e in range(EV):
                    acc[row, pl.ds(e * 16, 16)] = vs[e]

            @pl.when(ci + NBUF < ch)
            def _():
                pltpu.async_copy(
                    table_hbm.at[xv.at[ci + NBUF]], buf.at[slot], sems[slot]
                )

        def outer(i, carry):
            for slot in range(NBUF):
                step(slot, i * NBUF + slot)
            return carry

        lax.fori_loop(0, ch // NBUF, outer, 0)
        pltpu.sync_copy(acc, out_hbm.at[wid])

    return run(xg, table, dummy)


def _tc_mlp(sums, lens, w1t, b1, w2t, b2):
    bh = sums.shape[0]

    def body(s_ref, l_ref, w1_ref, b1_ref, w2_ref, b2_ref, out_ref):
        rep = s_ref[:] / l_ref[:]
        h = jnp.dot(rep, w1_ref[:], preferred_element_type=jnp.float32)
        h = jnp.maximum(h + b1_ref[:], 0.0)
        out_ref[:] = (
            jnp.dot(h, w2_ref[:], preferred_element_type=jnp.float32) + b2_ref[:]
        )

    return pl.pallas_call(
        body,
        out_shape=jax.ShapeDtypeStruct((bh, O), jnp.float32),
    )(sums, lens, w1t, b1, w2t, b2)


def kernel(x, lengths, table, W1, b1, W2, b2):
    dummy = jnp.zeros((CSZ, E), jnp.float32)
    lens = lengths.astype(jnp.float32).reshape(B, 1)
    w1t, b1r = W1.T, b1.reshape(1, H)
    w2t, b2r = W2.T, b2.reshape(1, O)
    nsp = 2
    bh = B // nsp
    outs = []
    for i in range(nsp):
        xh = lax.slice_in_dim(x, i * bh, (i + 1) * bh, axis=0)
        sums = _sc_gather_sum(xh, table, dummy).reshape(bh, E)
        lh = lax.slice_in_dim(lens, i * bh, (i + 1) * bh, axis=0)
        outs.append(_tc_mlp(sums, lh, w1t, b1r, w2t, b2r))
    return jnp.concatenate(outs, axis=0)


# speedup vs baseline: 1.0390x; 1.0390x over previous
import functools

import jax
import jax.numpy as jnp
from jax import lax
from jax.experimental import pallas as pl
from jax.experimental.pallas import tpu as pltpu
from jax.experimental.pallas import tpu_sc as plsc

B, S, E = 4096, 50, 128
H, O = 64, 10
NC, NS = 2, 16
NW = NC * NS
BPW = B // NW
PAIR = 1
CSZ = PAIR * S
CH = BPW // PAIR
EV = E // 16
NBUF = 8


def _rowsum(bufslot, base):
    UNR = 5

    def jb(j, carry):
        r = base + j * UNR
        vs = list(carry)
        for u in range(UNR):
            for e in range(EV):
                vs[e] = vs[e] + bufslot[r + u, pl.ds(e * 16, 16)]
        return tuple(vs)

    z = tuple(jnp.zeros((16,), jnp.float32) for _ in range(EV))
    return lax.fori_loop(0, S // UNR, jb, z)


def _sc_gather_sum(xg, table, dummy):
    bh = xg.shape[0]
    bpw = bh // NW
    ch = bpw // PAIR
    mesh = plsc.VectorSubcoreMesh(core_axis_name="c", subcore_axis_name="s")

    @functools.partial(
        pl.kernel,
        out_type=jax.ShapeDtypeStruct((NW, bpw, E), jnp.float32),
        mesh=mesh,
        scratch_types=[
            pltpu.VMEM((ch, CSZ), jnp.int32),
            pltpu.VMEM((NBUF, CSZ, E), jnp.float32),
            pltpu.VMEM((bpw, E), jnp.float32),
            [pltpu.SemaphoreType.DMA] * NBUF,
        ],
    )
    def run(x_hbm, table_hbm, dummy_hbm, out_hbm, xv, buf, acc, sems):
        wid = lax.axis_index("s") * NC + lax.axis_index("c")
        pltpu.sync_copy(x_hbm.at[pl.ds(wid * bpw, bpw)], xv)
        for slot in range(NBUF):
            pltpu.async_copy(table_hbm.at[xv.at[slot]], buf.at[slot], sems[slot])

        def step(slot, ci):
            pltpu.make_async_copy(
                dummy_hbm, buf.at[slot], sems[slot]
            ).wait()
            for p in range(PAIR):
                vs = _rowsum(buf.at[slot], p * S)
                row = ci * PAIR + p
                for e in range(EV):
                    acc[row, pl.ds(e * 16, 16)] = vs[e]

            @pl.when(ci + NBUF < ch)
            def _():
                pltpu.async_copy(
                    table_hbm.at[xv.at[ci + NBUF]], buf.at[slot], sems[slot]
                )

        def outer(i, carry):
            for slot in range(NBUF):
                step(slot, i * NBUF + slot)
            return carry

        lax.fori_loop(0, ch // NBUF, outer, 0)
        pltpu.sync_copy(acc, out_hbm.at[wid])

    return run(xg, table, dummy)


def _tc_mlp(sums, lens, w1t, b1, w2t, b2):
    bh = sums.shape[0]

    blk = 512

    def body(s_ref, l_ref, w1_ref, b1_ref, w2_ref, b2_ref, out_ref):
        rep = s_ref[:] / l_ref[:]
        h = jnp.dot(rep, w1_ref[:], preferred_element_type=jnp.float32)
        h = jnp.maximum(h + b1_ref[:], 0.0)
        out_ref[:] = (
            jnp.dot(h, w2_ref[:], preferred_element_type=jnp.float32) + b2_ref[:]
        )

    return pl.pallas_call(
        body,
        grid=(bh // blk,),
        in_specs=[
            pl.BlockSpec((blk, E), lambda i: (i, 0)),
            pl.BlockSpec((blk, 1), lambda i: (i, 0)),
            pl.BlockSpec((E, H), lambda i: (0, 0)),
            pl.BlockSpec((1, H), lambda i: (0, 0)),
            pl.BlockSpec((H, O), lambda i: (0, 0)),
            pl.BlockSpec((1, O), lambda i: (0, 0)),
        ],
        out_specs=pl.BlockSpec((blk, O), lambda i: (i, 0)),
        out_shape=jax.ShapeDtypeStruct((bh, O), jnp.float32),
    )(sums, lens, w1t, b1, w2t, b2)


def kernel(x, lengths, table, W1, b1, W2, b2):
    dummy = jnp.zeros((CSZ, E), jnp.float32)
    lens = lengths.astype(jnp.float32).reshape(B, 1)
    sums = _sc_gather_sum(x, table, dummy).reshape(B, E)
    return _tc_mlp(sums, lens, W1.T, b1.reshape(1, H), W2.T, b2.reshape(1, O))


# speedup vs baseline: 1.0630x; 1.0231x over previous
import functools

import jax
import jax.numpy as jnp
from jax import lax
from jax.experimental import pallas as pl
from jax.experimental.pallas import tpu as pltpu
from jax.experimental.pallas import tpu_sc as plsc

B, S, E = 4096, 50, 128
H, O = 64, 10
NC, NS = 2, 16
NW = NC * NS
BPW = B // NW
PAIR = 1
CSZ = PAIR * S
CH = BPW // PAIR
EV = E // 16
NBUF = 8


def _rowsum(bufslot, base):
    UNR = 5

    def jb(j, carry):
        r = base + j * UNR
        vs = list(carry)
        for u in range(UNR):
            for e in range(EV):
                vs[e] = vs[e] + bufslot[r + u, pl.ds(e * 16, 16)]
        return tuple(vs)

    z = tuple(jnp.zeros((16,), jnp.float32) for _ in range(EV))
    return lax.fori_loop(0, S // UNR, jb, z)


def _sc_gather_sum(xg, table, dummy):
    bh = xg.shape[0]
    bpw = bh // NW
    ch = bpw // PAIR
    mesh = plsc.VectorSubcoreMesh(core_axis_name="c", subcore_axis_name="s")

    @functools.partial(
        pl.kernel,
        out_type=jax.ShapeDtypeStruct((NW, bpw, E), jnp.float32),
        mesh=mesh,
        scratch_types=[
            pltpu.VMEM((ch, CSZ), jnp.int32),
            pltpu.VMEM((NBUF, CSZ, E), jnp.float32),
            pltpu.VMEM((bpw, E), jnp.float32),
            [pltpu.SemaphoreType.DMA] * NBUF,
            pltpu.SemaphoreType.DMA,
        ],
    )
    def run(x_hbm, table_hbm, dummy_hbm, out_hbm, xv, buf, acc, sems, osem):
        wid = lax.axis_index("s") * NC + lax.axis_index("c")
        pltpu.sync_copy(
            x_hbm.at[pl.ds(wid * bpw, NBUF)], xv.at[pl.ds(0, NBUF)]
        )
        for slot in range(NBUF):
            pltpu.async_copy(table_hbm.at[xv.at[slot]], buf.at[slot], sems[slot])
        pltpu.sync_copy(
            x_hbm.at[pl.ds(wid * bpw + NBUF, bpw - NBUF)],
            xv.at[pl.ds(NBUF, bpw - NBUF)],
        )

        def step(slot, ci):
            pltpu.make_async_copy(
                dummy_hbm, buf.at[slot], sems[slot]
            ).wait()
            for p in range(PAIR):
                vs = _rowsum(buf.at[slot], p * S)
                row = ci * PAIR + p
                for e in range(EV):
                    acc[row, pl.ds(e * 16, 16)] = vs[e]

            @pl.when(ci + NBUF < ch)
            def _():
                pltpu.async_copy(
                    table_hbm.at[xv.at[ci + NBUF]], buf.at[slot], sems[slot]
                )

        def outer(i, carry):
            for slot in range(NBUF):
                step(slot, i * NBUF + slot)
            return carry

        groups = ch // NBUF
        half = (groups // 2) * NBUF * PAIR
        lax.fori_loop(0, groups // 2, outer, 0)
        first = pltpu.async_copy(
            acc.at[pl.ds(0, half)], out_hbm.at[wid, pl.ds(0, half)], osem
        )
        lax.fori_loop(groups // 2, groups, outer, 0)
        pltpu.async_copy(
            acc.at[pl.ds(half, bpw - half)],
            out_hbm.at[wid, pl.ds(half, bpw - half)],
            osem,
        ).wait()
        first.wait()

    return run(xg, table, dummy)


def _tc_mlp(sums, lens, w1t, b1, w2t, b2):
    bh = sums.shape[0]

    def body(s_ref, l_ref, w1_ref, b1_ref, w2_ref, b2_ref, out_ref):
        rep = s_ref[:] / l_ref[:]
        h = jnp.dot(rep, w1_ref[:], preferred_element_type=jnp.float32)
        h = jnp.maximum(h + b1_ref[:], 0.0)
        out_ref[:] = (
            jnp.dot(h, w2_ref[:], preferred_element_type=jnp.float32) + b2_ref[:]
        )

    return pl.pallas_call(
        body,
        out_shape=jax.ShapeDtypeStruct((bh, O), jnp.float32),
    )(sums, lens, w1t, b1, w2t, b2)


def kernel(x, lengths, table, W1, b1, W2, b2):
    dummy = jnp.zeros((CSZ, E), jnp.float32)
    lens = lengths.astype(jnp.float32).reshape(B, 1)
    sums = _sc_gather_sum(x, table, dummy).reshape(B, E)
    return _tc_mlp(sums, lens, W1.T, b1.reshape(1, H), W2.T, b2.reshape(1, O))


# speedup vs baseline: 1.0911x; 1.0264x over previous
import functools

import jax
import jax.numpy as jnp
from jax import lax
from jax.experimental import pallas as pl
from jax.experimental.pallas import tpu as pltpu
from jax.experimental.pallas import tpu_sc as plsc

B, S, E = 4096, 50, 128
H, O = 64, 10
NC, NS = 2, 16
NW = NC * NS
BPW = B // NW
PAIR = 1
CSZ = PAIR * S
CH = BPW // PAIR
EV = E // 16
NBUF = 8


def _rowsum(bufslot, base):
    UNR = 5

    def jb(j, carry):
        r = base + j * UNR
        vs = list(carry)
        for u in range(UNR):
            for e in range(EV):
                vs[e] = vs[e] + bufslot[r + u, pl.ds(e * 16, 16)]
        return tuple(vs)

    z = tuple(jnp.zeros((16,), jnp.float32) for _ in range(EV))
    return lax.fori_loop(0, S // UNR, jb, z)


def _sc_gather_sum(xg, table, dummy):
    bh = xg.shape[0]
    bpw = bh // NW
    ch = bpw // PAIR
    mesh = plsc.VectorSubcoreMesh(core_axis_name="c", subcore_axis_name="s")

    @functools.partial(
        pl.kernel,
        out_type=jax.ShapeDtypeStruct((NW, bpw, E), jnp.float32),
        mesh=mesh,
        scratch_types=[
            pltpu.VMEM((ch, CSZ), jnp.int32),
            pltpu.VMEM((NBUF, CSZ, E), jnp.float32),
            pltpu.VMEM((bpw, E), jnp.float32),
            [pltpu.SemaphoreType.DMA] * NBUF,
        ],
    )
    def run(x_hbm, table_hbm, dummy_hbm, out_hbm, xv, buf, acc, sems):
        wid = lax.axis_index("s") * NC + lax.axis_index("c")
        pltpu.sync_copy(x_hbm.at[pl.ds(wid * bpw, bpw)], xv)
        for slot in range(NBUF):
            pltpu.async_copy(table_hbm.at[xv.at[slot]], buf.at[slot], sems[slot])

        def step(slot, ci):
            pltpu.make_async_copy(
                dummy_hbm, buf.at[slot], sems[slot]
            ).wait()
            for p in range(PAIR):
                vs = _rowsum(buf.at[slot], p * S)
                row = ci * PAIR + p
                for e in range(EV):
                    acc[row, pl.ds(e * 16, 16)] = vs[e]

            @pl.when(ci + NBUF < ch)
            def _():
                pltpu.async_copy(
                    table_hbm.at[xv.at[ci + NBUF]], buf.at[slot], sems[slot]
                )

        def outer(i, carry):
            for slot in range(NBUF):
                step(slot, i * NBUF + slot)
            return carry

        lax.fori_loop(0, ch // NBUF, outer, 0)
        pltpu.sync_copy(acc, out_hbm.at[wid])

    return run(xg, table, dummy)


def _tc_mlp(sums, lens, w1t, b1, w2t, b2):
    bh = sums.shape[0]

    def body(s_ref, l_ref, w1_ref, b1_ref, w2_ref, b2_ref, out_ref):
        rep = s_ref[:] / l_ref[:]
        h = jnp.dot(rep, w1_ref[:], preferred_element_type=jnp.float32)
        h = jnp.maximum(h + b1_ref[:], 0.0)
        out_ref[:] = (
            jnp.dot(h, w2_ref[:], preferred_element_type=jnp.float32) + b2_ref[:]
        )

    return pl.pallas_call(
        body,
        out_shape=jax.ShapeDtypeStruct((bh, O), jnp.float32),
    )(sums, lens, w1t, b1, w2t, b2)


def kernel(x, lengths, table, W1, b1, W2, b2):
    dummy = jnp.zeros((CSZ, E), jnp.float32)
    lens = lengths.astype(jnp.float32).reshape(B, 1)
    sums = _sc_gather_sum(x, table, dummy).reshape(B, E)
    return _tc_mlp(sums, lens, W1.T, b1.reshape(1, H), W2.T, b2.reshape(1, O))


# speedup vs baseline: 1.0916x; 1.0005x over previous
import functools

import jax
import jax.numpy as jnp
from jax import lax
from jax.experimental import pallas as pl
from jax.experimental.pallas import tpu as pltpu
from jax.experimental.pallas import tpu_sc as plsc

B, S, E = 4096, 50, 128
H, O = 64, 10
NC, NS = 2, 16
NW = NC * NS
BPW = B // NW
PAIR = 1
CSZ = PAIR * S
EV = E // 16
NBUF = 8


def _rowsum(bufslot, base):
    UNR = 5

    def jb(j, carry):
        r = base + j * UNR
        vs = list(carry)
        for u in range(UNR):
            for e in range(EV):
                vs[e] = vs[e] + bufslot[r + u, pl.ds(e * 16, 16)]
        return tuple(vs)

    z = tuple(jnp.zeros((16,), jnp.float32) for _ in range(EV))
    return lax.fori_loop(0, S // UNR, jb, z)


def _sc_gather_sum(xg, table, dummy):
    bh = xg.shape[0]
    bpw = bh // NW
    ch = bpw // PAIR
    mesh = plsc.VectorSubcoreMesh(core_axis_name="c", subcore_axis_name="s")

    @functools.partial(
        pl.kernel,
        out_type=jax.ShapeDtypeStruct((NW, bpw, E), jnp.float32),
        mesh=mesh,
        scratch_types=[
            pltpu.VMEM((ch, CSZ), jnp.int32),
            pltpu.VMEM((NBUF, CSZ, E), jnp.float32),
            pltpu.VMEM((bpw, E), jnp.float32),
            [pltpu.SemaphoreType.DMA] * NBUF,
        ],
    )
    def run(x_hbm, table_hbm, dummy_hbm, out_hbm, xv, buf, acc, sems):
        wid = lax.axis_index("s") * NC + lax.axis_index("c")
        pltpu.sync_copy(x_hbm.at[pl.ds(wid * bpw, bpw)], xv)
        for slot in range(NBUF):
            pltpu.async_copy(table_hbm.at[xv.at[slot]], buf.at[slot], sems[slot])

        def step(slot, ci):
            pltpu.make_async_copy(
                dummy_hbm, buf.at[slot], sems[slot]
            ).wait()
            for p in range(PAIR):
                vs = _rowsum(buf.at[slot], p * S)
                row = ci * PAIR + p
                for e in range(EV):
                    acc[row, pl.ds(e * 16, 16)] = vs[e]

            @pl.when(ci + NBUF < ch)
            def _():
                pltpu.async_copy(
                    table_hbm.at[xv.at[ci + NBUF]], buf.at[slot], sems[slot]
                )

        def outer(i, carry):
            for slot in range(NBUF):
                step(slot, i * NBUF + slot)
            return carry

        lax.fori_loop(0, ch // NBUF, outer, 0)
        pltpu.sync_copy(acc, out_hbm.at[wid])

    return run(xg, table, dummy)


def _tc_mlp(sums, lens, w1t, b1, w2t, b2):
    bh = sums.shape[0]

    def body(s_ref, l_ref, w1_ref, b1_ref, w2_ref, b2_ref, out_ref):
        rep = s_ref[:] / l_ref[:]
        h = jnp.dot(rep, w1_ref[:], preferred_element_type=jnp.float32)
        h = jnp.maximum(h + b1_ref[:], 0.0)
        out_ref[:] = (
            jnp.dot(h, w2_ref[:], preferred_element_type=jnp.float32) + b2_ref[:]
        )

    return pl.pallas_call(
        body,
        out_shape=jax.ShapeDtypeStruct((bh, O), jnp.float32),
    )(sums, lens, w1t, b1, w2t, b2)


def kernel(x, lengths, table, W1, b1, W2, b2):
    dummy = jnp.zeros((CSZ, E), jnp.float32)
    lens = lengths.astype(jnp.float32).reshape(B, 1)
    sums = _sc_gather_sum(x, table, dummy).reshape(B, E)
    return _tc_mlp(sums, lens, W1.T, b1.reshape(1, H), W2.T, b2.reshape(1, O))
